# Initial kernel scaffold; baseline (speedup 1.0000x reference)
import jax
import jax.numpy as jnp
from jax.experimental import pallas as pl


def kernel(inputs, gate_w, expert_w, expert_b):
    raise NotImplementedError("write your pallas kernel here")


# speedup vs baseline: 1.7327x; 1.7327x over previous
import functools

import jax
import jax.numpy as jnp
from jax import lax
from jax.experimental import pallas as pl
from jax.experimental.pallas import tpu as pltpu

E = 64
TOPK = 2
H = 768
T = 2048
B = 16

GATE_TBLK = 256
MM_TBLK = 256


def _gate_body(x_ref, gw_ref, sel_ref, wts_ref, acc_ref):
    i = pl.program_id(0)

    @pl.when(i == 0)
    def _():
        acc_ref[...] = jnp.zeros_like(acc_ref)

    acc_ref[...] += jnp.sum(x_ref[...], axis=0)

    @pl.when(i == pl.num_programs(0) - 1)
    def _():
        pooled = acc_ref[...] * (1.0 / T)
        logits = lax.dot_general(
            pooled, gw_ref[...], (((1,), (1,)), ((), ())),
            preferred_element_type=jnp.float32)
        iota = lax.broadcasted_iota(jnp.int32, (B, E), 1)
        m1 = jnp.max(logits, axis=1, keepdims=True)
        idx1 = jnp.min(jnp.where(logits == m1, iota, E), axis=1, keepdims=True)
        masked = jnp.where(iota == idx1, -jnp.inf, logits)
        m2 = jnp.max(masked, axis=1, keepdims=True)
        idx2 = jnp.min(jnp.where(masked == m2, iota, E), axis=1, keepdims=True)
        d = jnp.exp(m2 - m1)
        w1 = 1.0 / (1.0 + d)
        sel_ref[:, 0] = idx1[:, 0]
        sel_ref[:, 1] = idx2[:, 0]
        wts_ref[:, 0] = w1[:, 0]
        wts_ref[:, 1] = 1.0 - w1[:, 0]


def _routing(inputs):
    n_blk = T // GATE_TBLK
    return pl.pallas_call(
        _gate_body,
        grid=(n_blk,),
        in_specs=[
            pl.BlockSpec((GATE_TBLK, B, H), lambda i: (i, 0, 0)),
            pl.BlockSpec((E, H), lambda i: (0, 0)),
        ],
        out_specs=[
            pl.BlockSpec((B, TOPK), lambda i: (0, 0)),
            pl.BlockSpec((B, TOPK), lambda i: (0, 0)),
        ],
        out_shape=[
            jax.ShapeDtypeStruct((B, TOPK), jnp.int32),
            jax.ShapeDtypeStruct((B, TOPK), jnp.float32),
        ],
        scratch_shapes=[pltpu.VMEM((B, H), jnp.float32)],
    )


def _mm_body(sel_ref, wts_ref, x_ref, w0_ref, w1_ref, b0_ref, b1_ref,
             out_ref, wc_ref):
    b = pl.program_id(0)
    t = pl.program_id(1)
    w0 = wts_ref[b, 0]
    w1 = wts_ref[b, 1]

    @pl.when(t == 0)
    def _():
        wc_ref[...] = w0 * w0_ref[0] + w1 * w1_ref[0]

    y = lax.dot_general(
        x_ref[...], wc_ref[...], (((1,), (1,)), ((), ())),
        preferred_element_type=jnp.float32)
    bias = w0 * b0_ref[0, 0] + w1 * b1_ref[0, 0]
    out_ref[...] = y + bias[None, :]


def _expert_mm(x2, expert_w, expert_b, sel, wts):
    n_t = T // MM_TBLK
    grid_spec = pltpu.PrefetchScalarGridSpec(
        num_scalar_prefetch=2,
        grid=(B, n_t),
        in_specs=[
            pl.BlockSpec((MM_TBLK, H), lambda b, t, sel, wts: (t, b)),
            pl.BlockSpec((1, H, H), lambda b, t, sel, wts: (sel[b, 0], 0, 0)),
            pl.BlockSpec((1, H, H), lambda b, t, sel, wts: (sel[b, 1], 0, 0)),
            pl.BlockSpec((1, 1, H), lambda b, t, sel, wts: (sel[b, 0], 0, 0)),
            pl.BlockSpec((1, 1, H), lambda b, t, sel, wts: (sel[b, 1], 0, 0)),
        ],
        out_specs=pl.BlockSpec((MM_TBLK, H), lambda b, t, sel, wts: (t, b)),
        scratch_shapes=[pltpu.VMEM((H, H), jnp.float32)],
    )
    call = pl.pallas_call(
        _mm_body,
        grid_spec=grid_spec,
        out_shape=jax.ShapeDtypeStruct((T, B * H), jnp.float32),
        compiler_params=pltpu.CompilerParams(
            dimension_semantics=("arbitrary", "arbitrary")),
    )
    eb3 = expert_b.reshape(E, 1, H)
    return call(sel, wts, x2, expert_w, expert_w, eb3, eb3)


@jax.jit
def kernel(inputs, gate_w, expert_w, expert_b):
    sel, wts = _routing(inputs)(inputs, gate_w)
    x2 = inputs.reshape(T, B * H)
    out2 = _expert_mm(x2, expert_w, expert_b, sel, wts)
    return out2.reshape(T, B, H)


# speedup vs baseline: 2.8701x; 1.6565x over previous
import jax
import jax.numpy as jnp
from jax import lax
from jax.experimental import pallas as pl
from jax.experimental.pallas import tpu as pltpu

E = 64
TOPK = 2
H = 768
T = 2048
B = 16

GATE_TBLK = 256
MM_TBLK = 128
BG = 8


def _gate_body(x_ref, gw_ref, eb_ref, sel_ref, wts_ref, bias_ref, acc_ref):
    i = pl.program_id(0)

    @pl.when(i == 0)
    def _():
        acc_ref[...] = jnp.zeros_like(acc_ref)

    acc_ref[...] += jnp.sum(x_ref[...], axis=0)

    @pl.when(i == pl.num_programs(0) - 1)
    def _():
        pooled = acc_ref[...] * (1.0 / T)
        logits = lax.dot_general(
            pooled, gw_ref[...], (((1,), (1,)), ((), ())),
            preferred_element_type=jnp.float32)
        iota = lax.broadcasted_iota(jnp.int32, (B, E), 1)
        m1 = jnp.max(logits, axis=1, keepdims=True)
        idx1 = jnp.min(jnp.where(logits == m1, iota, E), axis=1, keepdims=True)
        masked = jnp.where(iota == idx1, -jnp.inf, logits)
        m2 = jnp.max(masked, axis=1, keepdims=True)
        idx2 = jnp.min(jnp.where(masked == m2, iota, E), axis=1, keepdims=True)
        d = jnp.exp(m2 - m1)
        w1 = 1.0 / (1.0 + d)
        w2 = 1.0 - w1
        sel_ref[:, 0] = idx1[:, 0]
        sel_ref[:, 1] = idx2[:, 0]
        wts_ref[:, 0] = w1[:, 0]
        wts_ref[:, 1] = w2[:, 0]
        onehot = jnp.where(iota == idx1, w1, 0.0) + jnp.where(iota == idx2, w2, 0.0)
        bias_ref[...] = lax.dot_general(
            onehot, eb_ref[...], (((1,), (0,)), ((), ())),
            preferred_element_type=jnp.float32)


def _routing(inputs, gate_w, expert_b):
    n_blk = T // GATE_TBLK
    return pl.pallas_call(
        _gate_body,
        grid=(n_blk,),
        in_specs=[
            pl.BlockSpec((GATE_TBLK, B, H), lambda i: (i, 0, 0)),
            pl.BlockSpec((E, H), lambda i: (0, 0)),
            pl.BlockSpec((E, H), lambda i: (0, 0)),
        ],
        out_specs=[
            pl.BlockSpec((B, TOPK), lambda i: (0, 0)),
            pl.BlockSpec((B, TOPK), lambda i: (0, 0)),
            pl.BlockSpec((B, H), lambda i: (0, 0)),
        ],
        out_shape=[
            jax.ShapeDtypeStruct((B, TOPK), jnp.int32),
            jax.ShapeDtypeStruct((B, TOPK), jnp.float32),
            jax.ShapeDtypeStruct((B, H), jnp.float32),
        ],
        scratch_shapes=[pltpu.VMEM((B, H), jnp.float32)],
    )(inputs, gate_w, expert_b)


def _mm_body(sel_ref, wts_ref, x_ref, ew_ref, bias_ref, out_ref,
             wc_ref, stage_ref, sem_ref):
    g = pl.program_id(0)
    t = pl.program_id(1)

    def _issue(slot, i):
        b = g * BG + i
        for k in range(TOPK):
            pltpu.make_async_copy(
                ew_ref.at[sel_ref[b, k]],
                stage_ref.at[slot, k],
                sem_ref.at[slot, k],
            ).start()

    @pl.when(t == 0)
    def _():
        _issue(0, 0)
        for i in range(BG):
            if i + 1 < BG:
                _issue((i + 1) % 2, i + 1)
            b = g * BG + i
            slot = i % 2
            for k in range(TOPK):
                pltpu.make_async_copy(
                    ew_ref.at[sel_ref[b, k]],
                    stage_ref.at[slot, k],
                    sem_ref.at[slot, k],
                ).wait()
            wc_ref[i] = (wts_ref[b, 0] * stage_ref[slot, 0]
                         + wts_ref[b, 1] * stage_ref[slot, 1])

    for i in range(BG):
        y = lax.dot_general(
            x_ref[:, i, :], wc_ref[i], (((1,), (1,)), ((), ())),
            preferred_element_type=jnp.float32)
        out_ref[:, i, :] = y + bias_ref[i][None, :]


def _expert_mm(inputs, expert_w, bias_c, sel, wts):
    n_t = T // MM_TBLK
    grid_spec = pltpu.PrefetchScalarGridSpec(
        num_scalar_prefetch=2,
        grid=(B // BG, n_t),
        in_specs=[
            pl.BlockSpec((MM_TBLK, BG, H), lambda g, t, sel, wts: (t, g, 0)),
            pl.BlockSpec(memory_space=pl.ANY),
            pl.BlockSpec((BG, H), lambda g, t, sel, wts: (g, 0)),
        ],
        out_specs=pl.BlockSpec((MM_TBLK, BG, H), lambda g, t, sel, wts: (t, g, 0)),
        scratch_shapes=[
            pltpu.VMEM((BG, H, H), jnp.float32),
            pltpu.VMEM((2, TOPK, H, H), jnp.float32),
            pltpu.SemaphoreType.DMA((2, TOPK)),
        ],
    )
    return pl.pallas_call(
        _mm_body,
        grid_spec=grid_spec,
        out_shape=jax.ShapeDtypeStruct((T, B, H), jnp.float32),
        compiler_params=pltpu.CompilerParams(
            dimension_semantics=("arbitrary", "arbitrary")),
    )(sel, wts, inputs, expert_w, bias_c)


@jax.jit
def kernel(inputs, gate_w, expert_w, expert_b):
    sel, wts, bias_c = _routing(inputs, gate_w, expert_b)
    return _expert_mm(inputs, expert_w, bias_c, sel, wts)
